# baseline (speedup 1.0000x reference)
import functools

import jax
import jax.numpy as jnp
from jax import lax
from jax.experimental import pallas as pl
from jax.experimental.pallas import tpu as pltpu
from jax.experimental.pallas import tpu_sc as plsc

_TOKENS = 16384
_HIDDEN = 2048
_EXPERTS = 64
_BT = 2048

_NC, _NS, _L = 2, 16, 16
_NW = _NC * _NS
_TPW = _TOKENS // _NW
_G = _TPW // _L


def _logits_body(x_ref, w_ref, b_ref, t_ref, logits_ref):
    x = x_ref[...]
    w = w_ref[...]
    logits = jnp.dot(x, w, preferred_element_type=jnp.float32)
    scale = 1.0 / jnp.abs(t_ref[0, 0])
    logits_ref[...] = (logits + b_ref[...]) * scale


def _tc_logits(hidden_states, gate_w, gate_b, temperature):
    grid = (_TOKENS // _BT,)
    return pl.pallas_call(
        _logits_body,
        grid=grid,
        in_specs=[
            pl.BlockSpec((_BT, _HIDDEN), lambda i: (i, 0)),
            pl.BlockSpec((_HIDDEN, _EXPERTS), lambda i: (0, 0)),
            pl.BlockSpec((1, _EXPERTS), lambda i: (0, 0)),
            pl.BlockSpec((1, 1), lambda i: (0, 0)),
        ],
        out_specs=pl.BlockSpec((_BT, _EXPERTS), lambda i: (i, 0)),
        out_shape=jax.ShapeDtypeStruct((_TOKENS, _EXPERTS), jnp.float32),
    )(
        hidden_states,
        gate_w,
        gate_b.reshape(1, _EXPERTS),
        temperature.reshape(1, 1),
    )


@functools.partial(
    pl.kernel,
    mesh=plsc.VectorSubcoreMesh(core_axis_name="c", subcore_axis_name="s"),
    out_type=[
        jax.ShapeDtypeStruct((_TOKENS * 2,), jnp.float32),
        jax.ShapeDtypeStruct((_TOKENS * 2,), jnp.int32),
    ],
    scratch_types=[
        pltpu.VMEM((_TPW * _EXPERTS,), jnp.float32),
        pltpu.VMEM((_TPW * 2,), jnp.float32),
        pltpu.VMEM((_TPW * 2,), jnp.int32),
    ],
    compiler_params=pltpu.CompilerParams(needs_layout_passes=False),
)
def _sc_route(logits_hbm, wout_hbm, eout_hbm, lg_v, w_v, e_v):
    wid = lax.axis_index("s") * _NC + lax.axis_index("c")
    base = wid * _TPW
    pltpu.sync_copy(logits_hbm.at[pl.ds(base * _EXPERTS, _TPW * _EXPERTS)], lg_v)

    lane = lax.iota(jnp.int32, _L)
    neg = jnp.full((_L,), -3.0e38, jnp.float32)
    zeros16 = jnp.zeros((_L,), jnp.int32)

    def group(g, _):
        rows = g * _L + lane
        flat0 = rows * _EXPERTS

        def estep(e, carry):
            m1, i1, m2, i2 = carry
            v = plsc.load_gather(lg_v, [flat0 + e])
            gt1 = v > m1
            gt2 = jnp.logical_and(v > m2, jnp.logical_not(gt1))
            e16 = jnp.full((_L,), e, jnp.int32)
            i2n = jnp.where(gt1, i1, jnp.where(gt2, e16, i2))
            m2n = jnp.where(gt1, m1, jnp.where(gt2, v, m2))
            i1n = jnp.where(gt1, e16, i1)
            m1n = jnp.where(gt1, v, m1)
            return m1n, i1n, m2n, i2n

        m1, i1, m2, i2 = lax.fori_loop(
            0, _EXPERTS, estep, (neg, zeros16, neg, zeros16), unroll=8
        )
        ex = jnp.exp(m2 - m1)
        den = 1.0 + ex
        pair0 = rows * 2
        plsc.store_scatter(w_v, [pair0], 1.0 / den)
        plsc.store_scatter(w_v, [pair0 + 1], ex / den)
        plsc.store_scatter(e_v, [pair0], i1)
        plsc.store_scatter(e_v, [pair0 + 1], i2)
        return 0

    lax.fori_loop(0, _G, group, 0)
    pltpu.sync_copy(w_v, wout_hbm.at[pl.ds(base * 2, _TPW * 2)])
    pltpu.sync_copy(e_v, eout_hbm.at[pl.ds(base * 2, _TPW * 2)])


def kernel(hidden_states, gate_w, gate_b, temperature, noise_w, noise_b):
    del noise_w, noise_b
    router_logits = _tc_logits(hidden_states, gate_w, gate_b, temperature)
    wflat, eflat = _sc_route(router_logits.reshape(-1))
    router_weights = wflat.reshape(_TOKENS, 2)
    select_experts = eflat.reshape(_TOKENS, 2)
    return (router_logits, router_weights, select_experts)
